# Initial kernel scaffold; baseline (speedup 1.0000x reference)
import functools

import jax
import jax.numpy as jnp
from jax.experimental import pallas as pl
from jax.experimental.pallas import tpu as pltpu

_B, _S, _D = 1, 2048, 1024
_E, _K, _F, _R = 8, 2, 512, 8
_T = _B * _S


def _merge_body(wg_ref, wu_ref, wd_ref, ag_ref, bg_ref, au_ref, bu_ref,
                ad_ref, bd_ref, wgt_ref, wut_ref, wdt_ref):
    def merged_t(w, b_, a_):
        lo = jax.lax.dot_general(a_, b_, (((0,), (1,)), ((), ())),
                                 preferred_element_type=jnp.float32)
        return (w.T + lo).astype(jnp.bfloat16)

    wgt_ref[0] = merged_t(wg_ref[0], bg_ref[0], ag_ref[0])
    wut_ref[0] = merged_t(wu_ref[0], bu_ref[0], au_ref[0])
    wdt_ref[0] = merged_t(wd_ref[0], bd_ref[0], ad_ref[0])


def _moe_body(x32_ref, x16_ref, wr_ref, wgt_ref, wut_ref, wdt_ref,
              out_ref, comb_ref):
    e = pl.program_id(0)

    @pl.when(e == 0)
    def _router():
        logits = jax.lax.dot_general(
            x32_ref[...], wr_ref[...], (((1,), (1,)), ((), ())),
            preferred_element_type=jnp.float32,
            precision=jax.lax.Precision.HIGHEST)
        m = jnp.max(logits, axis=-1, keepdims=True)
        p = jnp.exp(logits - m)
        p = p / jnp.sum(p, axis=-1, keepdims=True)
        lane = jax.lax.broadcasted_iota(jnp.int32, (_T, _E), 1)
        p1 = jnp.max(p, axis=-1, keepdims=True)
        i1 = jnp.min(jnp.where(p == p1, lane, _E), axis=-1, keepdims=True)
        m1 = lane == i1
        pr = jnp.where(m1, -1.0, p)
        p2 = jnp.max(pr, axis=-1, keepdims=True)
        i2 = jnp.min(jnp.where(pr == p2, lane, _E), axis=-1, keepdims=True)
        m2 = lane == i2
        denom = p1 + p2
        comb_ref[...] = (jnp.where(m1, p, 0.0) + jnp.where(m2, p, 0.0)) / denom

    x16 = x16_ref[...]
    g = jax.lax.dot_general(x16, wgt_ref[0], (((1,), (0,)), ((), ())),
                            preferred_element_type=jnp.float32)
    u = jax.lax.dot_general(x16, wut_ref[0], (((1,), (0,)), ((), ())),
                            preferred_element_type=jnp.float32)
    h = (g * (1.0 / (1.0 + jnp.exp(-g))) * u).astype(jnp.bfloat16)
    y = jax.lax.dot_general(h, wdt_ref[0], (((1,), (0,)), ((), ())),
                            preferred_element_type=jnp.float32)
    lane = jax.lax.broadcasted_iota(jnp.int32, (_T, _E), 1)
    c = jnp.sum(jnp.where(lane == e, comb_ref[...], 0.0), axis=-1,
                keepdims=True)
    contrib = y * c

    @pl.when(e == 0)
    def _init():
        out_ref[...] = contrib

    @pl.when(e != 0)
    def _acc():
        out_ref[...] += contrib


@functools.partial(jax.jit, static_argnames=("interpret",))
def kernel(hidden_states, Wr, Wg, Wu, Wd, Ag, Bg, Au, Bu, Ad, Bd,
           interpret=False):
    x = hidden_states.reshape(_T, _D)
    x16 = x.astype(jnp.bfloat16)

    wgt, wut, wdt = pl.pallas_call(
        _merge_body,
        grid=(_E,),
        in_specs=[
            pl.BlockSpec((1, _F, _D), lambda e: (e, 0, 0)),
            pl.BlockSpec((1, _F, _D), lambda e: (e, 0, 0)),
            pl.BlockSpec((1, _D, _F), lambda e: (e, 0, 0)),
            pl.BlockSpec((1, _R, _D), lambda e: (e, 0, 0)),
            pl.BlockSpec((1, _F, _R), lambda e: (e, 0, 0)),
            pl.BlockSpec((1, _R, _D), lambda e: (e, 0, 0)),
            pl.BlockSpec((1, _F, _R), lambda e: (e, 0, 0)),
            pl.BlockSpec((1, _R, _F), lambda e: (e, 0, 0)),
            pl.BlockSpec((1, _D, _R), lambda e: (e, 0, 0)),
        ],
        out_specs=[
            pl.BlockSpec((1, _D, _F), lambda e: (e, 0, 0)),
            pl.BlockSpec((1, _D, _F), lambda e: (e, 0, 0)),
            pl.BlockSpec((1, _F, _D), lambda e: (e, 0, 0)),
        ],
        out_shape=[
            jax.ShapeDtypeStruct((_E, _D, _F), jnp.bfloat16),
            jax.ShapeDtypeStruct((_E, _D, _F), jnp.bfloat16),
            jax.ShapeDtypeStruct((_E, _F, _D), jnp.bfloat16),
        ],
        interpret=interpret,
    )(Wg, Wu, Wd, Ag, Bg, Au, Bu, Ad, Bd)

    y = pl.pallas_call(
        _moe_body,
        grid=(_E,),
        in_specs=[
            pl.BlockSpec((_T, _D), lambda e: (0, 0)),
            pl.BlockSpec((_T, _D), lambda e: (0, 0)),
            pl.BlockSpec((_E, _D), lambda e: (0, 0)),
            pl.BlockSpec((1, _D, _F), lambda e: (e, 0, 0)),
            pl.BlockSpec((1, _D, _F), lambda e: (e, 0, 0)),
            pl.BlockSpec((1, _F, _D), lambda e: (e, 0, 0)),
        ],
        out_specs=pl.BlockSpec((_T, _D), lambda e: (0, 0)),
        out_shape=jax.ShapeDtypeStruct((_T, _D), jnp.float32),
        scratch_shapes=[pltpu.VMEM((_T, _E), jnp.float32)],
        interpret=interpret,
    )(x, x16, Wr, wgt, wut, wdt)

    return y.reshape(_B, _S, _D)


# speedup vs baseline: 2.2618x; 2.2618x over previous
import functools

import jax
import jax.numpy as jnp
from jax.experimental import pallas as pl
from jax.experimental.pallas import tpu as pltpu

_B, _S, _D = 1, 2048, 1024
_E, _K, _F, _R = 8, 2, 512, 8
_T = _B * _S


def _merge_body(wg_ref, wu_ref, wd_ref, ag_ref, bg_ref, au_ref, bu_ref,
                ad_ref, bd_ref, wgt_ref, wut_ref, wdt_ref):
    def merged_t(w, b_, a_):
        lo = jax.lax.dot_general(a_, b_, (((0,), (1,)), ((), ())),
                                 preferred_element_type=jnp.float32)
        return (w.T + lo).astype(jnp.bfloat16)

    wgt_ref[0] = merged_t(wg_ref[0], bg_ref[0], ag_ref[0])
    wut_ref[0] = merged_t(wu_ref[0], bu_ref[0], au_ref[0])
    wdt_ref[0] = merged_t(wd_ref[0], bd_ref[0], ad_ref[0])


def _moe_body(x16_ref, wr_ref, wgt_ref, wut_ref, wdt_ref,
              out_ref, comb_ref):
    e = pl.program_id(0)

    @pl.when(e == 0)
    def _router():
        logits = jax.lax.dot_general(
            x16_ref[...], wr_ref[...].astype(jnp.bfloat16),
            (((1,), (1,)), ((), ())),
            preferred_element_type=jnp.float32)
        m = jnp.max(logits, axis=-1, keepdims=True)
        p = jnp.exp(logits - m)
        p = p / jnp.sum(p, axis=-1, keepdims=True)
        lane = jax.lax.broadcasted_iota(jnp.int32, (_T, _E), 1)
        p1 = jnp.max(p, axis=-1, keepdims=True)
        i1 = jnp.min(jnp.where(p == p1, lane, _E), axis=-1, keepdims=True)
        m1 = lane == i1
        pr = jnp.where(m1, -1.0, p)
        p2 = jnp.max(pr, axis=-1, keepdims=True)
        i2 = jnp.min(jnp.where(pr == p2, lane, _E), axis=-1, keepdims=True)
        m2 = lane == i2
        denom = p1 + p2
        comb_ref[...] = (jnp.where(m1, p, 0.0) + jnp.where(m2, p, 0.0)) / denom

    x16 = x16_ref[...]
    g = jax.lax.dot_general(x16, wgt_ref[0], (((1,), (0,)), ((), ())),
                            preferred_element_type=jnp.float32)
    u = jax.lax.dot_general(x16, wut_ref[0], (((1,), (0,)), ((), ())),
                            preferred_element_type=jnp.float32)
    h = (g * (1.0 / (1.0 + jnp.exp(-g))) * u).astype(jnp.bfloat16)
    y = jax.lax.dot_general(h, wdt_ref[0], (((1,), (0,)), ((), ())),
                            preferred_element_type=jnp.float32)
    lane = jax.lax.broadcasted_iota(jnp.int32, (_T, _E), 1)
    c = jnp.sum(jnp.where(lane == e, comb_ref[...], 0.0), axis=-1,
                keepdims=True)
    contrib = y * c

    @pl.when(e == 0)
    def _init():
        out_ref[...] = contrib

    @pl.when(e != 0)
    def _acc():
        out_ref[...] += contrib


@functools.partial(jax.jit, static_argnames=("interpret",))
def kernel(hidden_states, Wr, Wg, Wu, Wd, Ag, Bg, Au, Bu, Ad, Bd,
           interpret=False):
    x = hidden_states.reshape(_T, _D)
    x16 = x.astype(jnp.bfloat16)

    wgt, wut, wdt = pl.pallas_call(
        _merge_body,
        grid=(_E,),
        in_specs=[
            pl.BlockSpec((1, _F, _D), lambda e: (e, 0, 0)),
            pl.BlockSpec((1, _F, _D), lambda e: (e, 0, 0)),
            pl.BlockSpec((1, _D, _F), lambda e: (e, 0, 0)),
            pl.BlockSpec((1, _R, _D), lambda e: (e, 0, 0)),
            pl.BlockSpec((1, _F, _R), lambda e: (e, 0, 0)),
            pl.BlockSpec((1, _R, _D), lambda e: (e, 0, 0)),
            pl.BlockSpec((1, _F, _R), lambda e: (e, 0, 0)),
            pl.BlockSpec((1, _R, _F), lambda e: (e, 0, 0)),
            pl.BlockSpec((1, _D, _R), lambda e: (e, 0, 0)),
        ],
        out_specs=[
            pl.BlockSpec((1, _D, _F), lambda e: (e, 0, 0)),
            pl.BlockSpec((1, _D, _F), lambda e: (e, 0, 0)),
            pl.BlockSpec((1, _F, _D), lambda e: (e, 0, 0)),
        ],
        out_shape=[
            jax.ShapeDtypeStruct((_E, _D, _F), jnp.bfloat16),
            jax.ShapeDtypeStruct((_E, _D, _F), jnp.bfloat16),
            jax.ShapeDtypeStruct((_E, _F, _D), jnp.bfloat16),
        ],
        interpret=interpret,
    )(Wg, Wu, Wd, Ag, Bg, Au, Bu, Ad, Bd)

    y = pl.pallas_call(
        _moe_body,
        grid=(_E,),
        in_specs=[
            pl.BlockSpec((_T, _D), lambda e: (0, 0)),
            pl.BlockSpec((_E, _D), lambda e: (0, 0)),
            pl.BlockSpec((1, _D, _F), lambda e: (e, 0, 0)),
            pl.BlockSpec((1, _D, _F), lambda e: (e, 0, 0)),
            pl.BlockSpec((1, _F, _D), lambda e: (e, 0, 0)),
        ],
        out_specs=pl.BlockSpec((_T, _D), lambda e: (0, 0)),
        out_shape=jax.ShapeDtypeStruct((_T, _D), jnp.float32),
        scratch_shapes=[pltpu.VMEM((_T, _E), jnp.float32)],
        interpret=interpret,
    )(x16, Wr, wgt, wut, wdt)

    return y.reshape(_B, _S, _D)


# speedup vs baseline: 2.7515x; 1.2165x over previous
import functools

import jax
import jax.numpy as jnp
from jax.experimental import pallas as pl
from jax.experimental.pallas import tpu as pltpu

_B, _S, _D = 1, 2048, 1024
_E, _K, _F, _R = 8, 2, 512, 8
_T = _B * _S
_TC = 512


def _moe_body(x_ref, wr_ref, wg_ref, wu_ref, wd_ref, ag_ref, bg_ref,
              au_ref, bu_ref, ad_ref, bd_ref, out_ref, x16_ref, comb_ref):
    e = pl.program_id(0)

    @pl.when(e == 0)
    def _router():
        x16_ref[...] = x_ref[...].astype(jnp.bfloat16)
        logits = jax.lax.dot_general(
            x16_ref[...], wr_ref[...].astype(jnp.bfloat16),
            (((1,), (1,)), ((), ())),
            preferred_element_type=jnp.float32)
        m = jnp.max(logits, axis=-1, keepdims=True)
        p = jnp.exp(logits - m)
        p = p / jnp.sum(p, axis=-1, keepdims=True)
        lane = jax.lax.broadcasted_iota(jnp.int32, (_T, _E), 1)
        p1 = jnp.max(p, axis=-1, keepdims=True)
        i1 = jnp.min(jnp.where(p == p1, lane, _E), axis=-1, keepdims=True)
        m1 = lane == i1
        pr = jnp.where(m1, -1.0, p)
        p2 = jnp.max(pr, axis=-1, keepdims=True)
        i2 = jnp.min(jnp.where(pr == p2, lane, _E), axis=-1, keepdims=True)
        m2 = lane == i2
        comb_ref[...] = (jnp.where(m1, p, 0.0) + jnp.where(m2, p, 0.0)) / (p1 + p2)

    def merged_t(w, b_, a_):
        lo = jax.lax.dot_general(a_, b_, (((0,), (1,)), ((), ())),
                                 preferred_element_type=jnp.float32)
        return (w.T + lo).astype(jnp.bfloat16)

    wgt = merged_t(wg_ref[0], bg_ref[0], ag_ref[0])
    wut = merged_t(wu_ref[0], bu_ref[0], au_ref[0])
    wdt = merged_t(wd_ref[0], bd_ref[0], ad_ref[0])

    for c in range(_T // _TC):
        sl = pl.ds(c * _TC, _TC)
        xc = x16_ref[sl, :]
        g = jax.lax.dot_general(xc, wgt, (((1,), (0,)), ((), ())),
                                preferred_element_type=jnp.float32)
        u = jax.lax.dot_general(xc, wut, (((1,), (0,)), ((), ())),
                                preferred_element_type=jnp.float32)
        h = (g * (1.0 / (1.0 + jnp.exp(-g))) * u).astype(jnp.bfloat16)
        y = jax.lax.dot_general(h, wdt, (((1,), (0,)), ((), ())),
                                preferred_element_type=jnp.float32)
        lane = jax.lax.broadcasted_iota(jnp.int32, (_TC, _E), 1)
        cw = jnp.sum(jnp.where(lane == e, comb_ref[sl, :], 0.0), axis=-1,
                     keepdims=True)
        contrib = y * cw

        @pl.when(e == 0)
        def _init():
            out_ref[sl, :] = contrib

        @pl.when(e != 0)
        def _acc():
            out_ref[sl, :] += contrib


@functools.partial(jax.jit, static_argnames=("interpret",))
def kernel(hidden_states, Wr, Wg, Wu, Wd, Ag, Bg, Au, Bu, Ad, Bd,
           interpret=False):
    x = hidden_states.reshape(_T, _D)

    y = pl.pallas_call(
        _moe_body,
        grid=(_E,),
        in_specs=[
            pl.BlockSpec((_T, _D), lambda e: (0, 0)),
            pl.BlockSpec((_E, _D), lambda e: (0, 0)),
            pl.BlockSpec((1, _F, _D), lambda e: (e, 0, 0)),
            pl.BlockSpec((1, _F, _D), lambda e: (e, 0, 0)),
            pl.BlockSpec((1, _D, _F), lambda e: (e, 0, 0)),
            pl.BlockSpec((1, _R, _D), lambda e: (e, 0, 0)),
            pl.BlockSpec((1, _F, _R), lambda e: (e, 0, 0)),
            pl.BlockSpec((1, _R, _D), lambda e: (e, 0, 0)),
            pl.BlockSpec((1, _F, _R), lambda e: (e, 0, 0)),
            pl.BlockSpec((1, _R, _F), lambda e: (e, 0, 0)),
            pl.BlockSpec((1, _D, _R), lambda e: (e, 0, 0)),
        ],
        out_specs=pl.BlockSpec((_T, _D), lambda e: (0, 0)),
        out_shape=jax.ShapeDtypeStruct((_T, _D), jnp.float32),
        scratch_shapes=[pltpu.VMEM((_T, _D), jnp.bfloat16),
                        pltpu.VMEM((_T, _E), jnp.float32)],
        interpret=interpret,
    )(x, Wr, Wg, Wu, Wd, Ag, Bg, Au, Bu, Ad, Bd)

    return y.reshape(_B, _S, _D)


# speedup vs baseline: 3.0325x; 1.1021x over previous
import functools

import jax
import jax.numpy as jnp
from jax.experimental import pallas as pl
from jax.experimental.pallas import tpu as pltpu

_B, _S, _D = 1, 2048, 1024
_E, _K, _F, _R = 8, 2, 512, 8
_T = _B * _S
_TC = 512
_EF = _E * _F


def _moe_body(x16_ref, wr_ref, wg_ref, wu_ref, wd_ref, ag_ref, bg_ref,
              au_ref, bu_ref, ad_ref, bd_ref, out_ref,
              comb_ref, h_ref, wdall_ref, wgu_ref):
    e = pl.program_id(0)

    @pl.when(e == 0)
    def _router():
        logits = jax.lax.dot_general(
            x16_ref[...], wr_ref[...].astype(jnp.bfloat16),
            (((1,), (1,)), ((), ())),
            preferred_element_type=jnp.float32)
        m = jnp.max(logits, axis=-1, keepdims=True)
        p = jnp.exp(logits - m)
        p = p / jnp.sum(p, axis=-1, keepdims=True)
        lane = jax.lax.broadcasted_iota(jnp.int32, (_T, _E), 1)
        p1 = jnp.max(p, axis=-1, keepdims=True)
        i1 = jnp.min(jnp.where(p == p1, lane, _E), axis=-1, keepdims=True)
        m1 = lane == i1
        pr = jnp.where(m1, -1.0, p)
        p2 = jnp.max(pr, axis=-1, keepdims=True)
        i2 = jnp.min(jnp.where(pr == p2, lane, _E), axis=-1, keepdims=True)
        m2 = lane == i2
        comb_ref[...] = (jnp.where(m1, p, 0.0) + jnp.where(m2, p, 0.0)) / (p1 + p2)

    @pl.when(e < _E)
    def _expert():
        def merged(w, b_, a_):
            lo = jax.lax.dot_general(b_, a_, (((1,), (0,)), ((), ())),
                                     preferred_element_type=jnp.float32)
            return (w + lo).astype(jnp.bfloat16)

        col = pl.multiple_of(e * _F, _F)
        wgu_ref[0:_F, :] = merged(wg_ref[0], bg_ref[0], ag_ref[0])
        wgu_ref[_F:2 * _F, :] = merged(wu_ref[0], bu_ref[0], au_ref[0])
        wdall_ref[:, pl.ds(col, _F)] = merged(wd_ref[0], bd_ref[0], ad_ref[0])

        for c in range(_T // _TC):
            sl = pl.ds(c * _TC, _TC)
            gu = jax.lax.dot_general(x16_ref[sl, :], wgu_ref[...],
                                     (((1,), (1,)), ((), ())),
                                     preferred_element_type=jnp.float32)
            g = gu[:, :_F]
            u = gu[:, _F:]
            lane = jax.lax.broadcasted_iota(jnp.int32, (_TC, _E), 1)
            cw = jnp.sum(jnp.where(lane == e, comb_ref[sl, :], 0.0), axis=-1,
                         keepdims=True)
            h_ref[sl, pl.ds(col, _F)] = (
                g * (1.0 / (1.0 + jnp.exp(-g))) * u * cw).astype(jnp.bfloat16)

    @pl.when(e == _E)
    def _down():
        for c in range(_T // _TC):
            sl = pl.ds(c * _TC, _TC)
            out_ref[sl, :] = jax.lax.dot_general(
                h_ref[sl, :], wdall_ref[...], (((1,), (1,)), ((), ())),
                preferred_element_type=jnp.float32)


@functools.partial(jax.jit, static_argnames=("interpret",))
def kernel(hidden_states, Wr, Wg, Wu, Wd, Ag, Bg, Au, Bu, Ad, Bd,
           interpret=False):
    x16 = hidden_states.reshape(_T, _D).astype(jnp.bfloat16)

    def eb(e):
        ec = jnp.minimum(e, _E - 1)
        return ec

    y = pl.pallas_call(
        _moe_body,
        grid=(_E + 1,),
        in_specs=[
            pl.BlockSpec((_T, _D), lambda e: (0, 0)),
            pl.BlockSpec((_E, _D), lambda e: (0, 0)),
            pl.BlockSpec((1, _F, _D), lambda e: (eb(e), 0, 0)),
            pl.BlockSpec((1, _F, _D), lambda e: (eb(e), 0, 0)),
            pl.BlockSpec((1, _D, _F), lambda e: (eb(e), 0, 0)),
            pl.BlockSpec((1, _R, _D), lambda e: (eb(e), 0, 0)),
            pl.BlockSpec((1, _F, _R), lambda e: (eb(e), 0, 0)),
            pl.BlockSpec((1, _R, _D), lambda e: (eb(e), 0, 0)),
            pl.BlockSpec((1, _F, _R), lambda e: (eb(e), 0, 0)),
            pl.BlockSpec((1, _R, _F), lambda e: (eb(e), 0, 0)),
            pl.BlockSpec((1, _D, _R), lambda e: (eb(e), 0, 0)),
        ],
        out_specs=pl.BlockSpec((_T, _D), lambda e: (0, 0)),
        out_shape=jax.ShapeDtypeStruct((_T, _D), jnp.float32),
        scratch_shapes=[pltpu.VMEM((_T, _E), jnp.float32),
                        pltpu.VMEM((_T, _EF), jnp.bfloat16),
                        pltpu.VMEM((_D, _EF), jnp.bfloat16),
                        pltpu.VMEM((2 * _F, _D), jnp.bfloat16)],
        interpret=interpret,
    )(x16, Wr, Wg, Wu, Wd, Ag, Bg, Au, Bu, Ad, Bd)

    return y.reshape(_B, _S, _D)
